# Initial kernel scaffold; baseline (speedup 1.0000x reference)
import functools

import jax
import jax.numpy as jnp
from jax import lax
from jax.experimental import pallas as pl
from jax.experimental.pallas import tpu as pltpu
from jax.experimental.pallas import tpu_sc as plsc

_VOCAB = 1000000
_D = 32
_B_TOTAL = 16384 * 26
_NC, _NS = 2, 16
_NW = _NC * _NS
_BPW = _B_TOTAL // _NW
_CHUNK = 1024
_N_CHUNKS = _BPW // _CHUNK


@functools.cache
def _make_lookup():
    mesh = plsc.VectorSubcoreMesh(
        core_axis_name="c", subcore_axis_name="s",
        num_cores=_NC, num_subcores=_NS,
    )

    @functools.partial(
        pl.kernel,
        out_type=jax.ShapeDtypeStruct((_B_TOTAL, _D), jnp.float32),
        mesh=mesh,
        scratch_types=[
            pltpu.VMEM((_BPW,), jnp.int32),
            pltpu.VMEM((_CHUNK, _D), jnp.float32),
            pltpu.VMEM((_CHUNK, _D), jnp.float32),
            pltpu.SemaphoreType.DMA,
            pltpu.SemaphoreType.DMA,
            pltpu.SemaphoreType.DMA,
            pltpu.SemaphoreType.DMA,
        ],
    )
    def lookup(idx_hbm, table_hbm, out_hbm,
               idx_v, rows0, rows1, gsem0, gsem1, osem0, osem1):
        wid = lax.axis_index("s") * _NC + lax.axis_index("c")
        base = wid * _BPW
        pltpu.sync_copy(idx_hbm.at[pl.ds(base, _BPW)], idx_v)

        rows = (rows0, rows1)
        gsem = (gsem0, gsem1)
        osem = (osem0, osem1)
        gcp = [None, None]
        ocp = [None, None]
        for i in range(_N_CHUNKS):
            b = i & 1
            if ocp[b] is not None:
                ocp[b].wait()
            gcp[b] = pltpu.async_copy(
                table_hbm.at[idx_v.at[pl.ds(i * _CHUNK, _CHUNK)]],
                rows[b], gsem[b])
            if i > 0:
                pb = (i - 1) & 1
                gcp[pb].wait()
                ocp[pb] = pltpu.async_copy(
                    rows[pb],
                    out_hbm.at[pl.ds(base + (i - 1) * _CHUNK, _CHUNK)],
                    osem[pb])
        last = (_N_CHUNKS - 1) & 1
        gcp[last].wait()
        ocp[last] = pltpu.async_copy(
            rows[last],
            out_hbm.at[pl.ds(base + (_N_CHUNKS - 1) * _CHUNK, _CHUNK)],
            osem[last])
        ocp[1 - last].wait()
        ocp[last].wait()

    return lookup


@jax.jit
def kernel(input, weight):
    idx = input.reshape(-1).astype(jnp.int32)
    out = _make_lookup()(idx, weight)
    return out.reshape(input.shape + (weight.shape[1],))


# speedup vs baseline: 1.5756x; 1.5756x over previous
import functools

import jax
import jax.numpy as jnp
from jax import lax
from jax.experimental import pallas as pl
from jax.experimental.pallas import tpu as pltpu
from jax.experimental.pallas import tpu_sc as plsc

_VOCAB = 1000000
_D = 32
_B_TOTAL = 16384 * 26
_NC, _NS = 2, 16
_NW = _NC * _NS
_BPW = _B_TOTAL // _NW
_CHUNK = 1024
_N_CHUNKS = _BPW // _CHUNK


@functools.cache
def _make_lookup():
    mesh = plsc.VectorSubcoreMesh(
        core_axis_name="c", subcore_axis_name="s",
        num_cores=_NC, num_subcores=_NS,
    )

    @functools.partial(
        pl.kernel,
        out_type=jax.ShapeDtypeStruct((_B_TOTAL, _D), jnp.float32),
        mesh=mesh,
        compiler_params=pltpu.CompilerParams(use_tc_tiling_on_sc=False),
        scratch_types=[
            pltpu.VMEM((_BPW,), jnp.int32),
            pltpu.VMEM((_CHUNK, _D), jnp.float32),
            pltpu.VMEM((_CHUNK, _D), jnp.float32),
            pltpu.SemaphoreType.DMA,
            pltpu.SemaphoreType.DMA,
            pltpu.SemaphoreType.DMA,
            pltpu.SemaphoreType.DMA,
        ],
    )
    def lookup(idx_hbm, table_hbm, out_hbm,
               idx_v, rows0, rows1, gsem0, gsem1, osem0, osem1):
        wid = lax.axis_index("s") * _NC + lax.axis_index("c")
        base = wid * _BPW
        pltpu.sync_copy(idx_hbm.at[pl.ds(base, _BPW)], idx_v)

        rows = (rows0, rows1)
        gsem = (gsem0, gsem1)
        osem = (osem0, osem1)
        gcp = [None, None]
        ocp = [None, None]
        for i in range(_N_CHUNKS):
            b = i & 1
            if ocp[b] is not None:
                ocp[b].wait()
            gcp[b] = pltpu.async_copy(
                table_hbm.at[idx_v.at[pl.ds(i * _CHUNK, _CHUNK)]],
                rows[b], gsem[b])
            if i > 0:
                pb = (i - 1) & 1
                gcp[pb].wait()
                ocp[pb] = pltpu.async_copy(
                    rows[pb],
                    out_hbm.at[pl.ds(base + (i - 1) * _CHUNK, _CHUNK)],
                    osem[pb])
        last = (_N_CHUNKS - 1) & 1
        gcp[last].wait()
        ocp[last] = pltpu.async_copy(
            rows[last],
            out_hbm.at[pl.ds(base + (_N_CHUNKS - 1) * _CHUNK, _CHUNK)],
            osem[last])
        ocp[1 - last].wait()
        ocp[last].wait()

    return lookup


@jax.jit
def kernel(input, weight):
    idx = input.reshape(-1).astype(jnp.int32)
    out = _make_lookup()(idx, weight)
    return out.reshape(input.shape + (weight.shape[1],))
